# baseline (speedup 1.0000x reference)
import jax
import jax.numpy as jnp
from jax.experimental import pallas as pl
from jax.experimental.pallas import tpu as pltpu

_SEQ_BLOCK = 1024


def _emb_kernel(inp_ref, wk_ref, wks_ref, bc_ref, bss_ref, out_ref):
    tab = wk_ref[...] * bc_ref[0] + wks_ref[...] * bss_ref[0]
    m = (inp_ref[...] != 0).astype(tab.dtype)
    out_ref[...] = tab[None, :, :] * m[:, :, None]


def kernel(input_tensor, weights):
    batch, seq_len = input_tensor.shape
    dim = weights.shape[1]
    s_blk = _SEQ_BLOCK if seq_len % _SEQ_BLOCK == 0 else seq_len
    n_blk = seq_len // s_blk

    wk = jax.lax.slice(weights, (1, 0), (1 + s_blk, dim))
    wks = wk.reshape(s_blk, dim // 2, 2)[:, :, ::-1].reshape(s_blk, dim)

    base = weights[jnp.arange(n_blk) * s_blk]
    base = base.at[0].set(jnp.tile(jnp.array([0.0, 1.0], weights.dtype),
                                   dim // 2))
    pairs = base.reshape(n_blk, dim // 2, 2)
    sin_b = pairs[:, :, 0:1]
    cos_b = pairs[:, :, 1:2]
    bc = jnp.broadcast_to(cos_b, (n_blk, dim // 2, 2)).reshape(n_blk, dim)
    sign = jnp.tile(jnp.array([1.0, -1.0], weights.dtype), dim // 2)
    bss = jnp.broadcast_to(sin_b, (n_blk, dim // 2, 2)).reshape(n_blk, dim)
    bss = bss * sign
    bc = bc.reshape(n_blk, 1, dim)
    bss = bss.reshape(n_blk, 1, dim)

    out = pl.pallas_call(
        _emb_kernel,
        grid=(n_blk,),
        in_specs=[
            pl.BlockSpec((batch, s_blk), lambda i: (0, i)),
            pl.BlockSpec((s_blk, dim), lambda i: (0, 0)),
            pl.BlockSpec((s_blk, dim), lambda i: (0, 0)),
            pl.BlockSpec((1, 1, dim), lambda i: (i, 0, 0)),
            pl.BlockSpec((1, 1, dim), lambda i: (i, 0, 0)),
        ],
        out_specs=pl.BlockSpec((batch, s_blk, dim), lambda i: (0, i, 0)),
        out_shape=jax.ShapeDtypeStruct((batch, seq_len, dim), weights.dtype),
        compiler_params=pltpu.CompilerParams(
            dimension_semantics=("arbitrary",),
        ),
    )(input_tensor, wk, wks, bc, bss)
    return out
